# baseline (speedup 1.0000x reference)
import functools

import jax
import jax.numpy as jnp
from jax.experimental import pallas as pl
from jax.experimental.pallas import tpu as pltpu


def _gem_planes_kernel(xa_ref, xb_ref, o_ref, *, hw, eps, inv_hw, inv_p):
    def half(x_ref):
        def body(i, acc):
            x = jnp.maximum(x_ref[i], jnp.float32(eps))
            return acc + x * x * x
        acc = jax.lax.fori_loop(
            0, hw, body, jnp.zeros(x_ref.shape[1:], jnp.float32), unroll=True)
        return jnp.power(acc * jnp.float32(inv_hw), jnp.float32(inv_p))

    ra = half(xa_ref).reshape(-1)
    rb = half(xb_ref).reshape(-1)
    o_ref[...] = jnp.concatenate([ra, rb])


def _gem(x, p=3.0, eps=1e-6):
    N, C, H, W = x.shape
    HW = H * W
    xt = jnp.transpose(x, (2, 3, 0, 1)).reshape(HW, N, C)

    bn = 16
    while N % bn != 0:
        bn //= 2
    grid = N // bn
    hn = bn // 2

    kernel_fn = functools.partial(
        _gem_planes_kernel, hw=HW, eps=float(eps), inv_hw=1.0 / float(HW),
        inv_p=1.0 / float(p))
    out = pl.pallas_call(
        kernel_fn,
        out_shape=jax.ShapeDtypeStruct((N * C,), x.dtype),
        grid=(grid,),
        in_specs=[
            pl.BlockSpec((HW, hn, C), lambda j: (0, 2 * j, 0)),
            pl.BlockSpec((HW, hn, C), lambda j: (0, 2 * j + 1, 0)),
        ],
        out_specs=pl.BlockSpec((bn * C,), lambda j: (j,)),
        compiler_params=pltpu.CompilerParams(
            dimension_semantics=("parallel",),
            vmem_limit_bytes=int(32 << 20)),
    )(xt, xt)
    return out.reshape(N, C, 1, 1)


def kernel(x):
    return _gem(x, p=3.0, eps=1e-6)


# speedup vs baseline: 1.0051x; 1.0051x over previous
import functools

import jax
import jax.numpy as jnp
from jax.experimental import pallas as pl
from jax.experimental.pallas import tpu as pltpu


def _gem_planes_kernel(x_ref, o_ref, *, hw, eps, inv_hw, inv_p):
    def body(i, acc):
        x = jnp.maximum(x_ref[i], jnp.float32(eps))
        return acc + x * x * x
    acc = jax.lax.fori_loop(
        0, hw, body, jnp.zeros(x_ref.shape[1:], jnp.float32), unroll=True)
    res = jnp.power(acc * jnp.float32(inv_hw), jnp.float32(inv_p))
    o_ref[...] = res.reshape(o_ref.shape)


def _gem(x, p=3.0, eps=1e-6):
    N, C, H, W = x.shape
    HW = H * W
    xt = jnp.transpose(x, (2, 3, 0, 1)).reshape(HW, N, C)

    bn = 16
    while N % bn != 0:
        bn //= 2
    grid = N // bn

    kernel_fn = functools.partial(
        _gem_planes_kernel, hw=HW, eps=float(eps), inv_hw=1.0 / float(HW),
        inv_p=1.0 / float(p))
    out = pl.pallas_call(
        kernel_fn,
        out_shape=jax.ShapeDtypeStruct((N * C,), x.dtype),
        grid=(grid,),
        in_specs=[pl.BlockSpec((HW, bn, C), lambda j: (0, j, 0))],
        out_specs=pl.BlockSpec((bn * C,), lambda j: (j,)),
        compiler_params=pltpu.CompilerParams(
            dimension_semantics=("parallel",),
            vmem_limit_bytes=int(32 << 20)),
    )(xt)
    return out.reshape(N, C, 1, 1)


def kernel(x):
    return _gem(x, p=3.0, eps=1e-6)
